# baseline (device time: 316428 ns/iter reference)
import jax
import jax.numpy as jnp
from jax import lax
from jax.experimental import pallas as pl
from jax.experimental.pallas import tpu as pltpu

N_DEV = 4


def kernel(x, w_mat):
    m, k_per = x.shape
    _, n = w_mat.shape
    ch = m // N_DEV
    nh = n // 2
    hh = ch // 2

    def body(x_ref, w_ref, oinit_ref, out_ref, xbuf, buf_a, buf_b, buf_c,
             comm_r, comm_l, send_r, recv_r, send_l, recv_l, cps, cpo):
        del oinit_ref
        my = lax.axis_index("i")
        left = lax.rem(my + N_DEV - 1, N_DEV)
        right = lax.rem(my + 1, N_DEV)

        def sub_rows(sub):
            return pl.ds(sub * hh, hh)

        def half_cols(half):
            return pl.ds(half * nh, nh)

        def out_at(c, half, sub):
            return out_ref.at[pl.ds(c * ch + sub * hh, hh), half_cols(half)]

        def rdma_r(src, dst, slot):
            return pltpu.make_async_remote_copy(
                src_ref=src, dst_ref=dst,
                send_sem=send_r.at[slot], recv_sem=recv_r.at[slot],
                device_id=(right,), device_id_type=pl.DeviceIdType.MESH,
            )

        def rdma_l(src, dst, slot):
            return pltpu.make_async_remote_copy(
                src_ref=src, dst_ref=dst,
                send_sem=send_l.at[slot], recv_sem=recv_l.at[slot],
                device_id=(left,), device_id_type=pl.DeviceIdType.MESH,
            )

        def load_x(c, slot):
            cp = pltpu.make_async_copy(
                x_ref.at[pl.ds(c * ch, ch), :], xbuf.at[slot], cps.at[slot])
            cp.start()
            return cp

        cp_a = load_x(my, 0)

        barrier_sem = pltpu.get_barrier_semaphore()
        for nbr in (left, right):
            pl.semaphore_signal(
                barrier_sem, inc=1,
                device_id=(nbr,), device_id_type=pl.DeviceIdType.MESH,
            )
        pl.semaphore_wait(barrier_sem, 2)

        cp_a.wait()
        cp_b = load_x(lax.rem(my - 1 + N_DEV, N_DEV), 1)
        rs0 = []
        for sub in range(2):
            buf_a[sub * hh:(sub + 1) * hh, :] = jnp.dot(
                xbuf[0, sub * hh:(sub + 1) * hh, :], w_ref[:, :],
                preferred_element_type=jnp.float32)
            r0 = rdma_r(buf_a.at[sub_rows(sub), half_cols(0)],
                        comm_r.at[0, sub_rows(sub), :], sub)
            l0 = rdma_l(buf_a.at[sub_rows(sub), half_cols(1)],
                        comm_l.at[0, sub_rows(sub), :], sub)
            r0.start()
            l0.start()
            rs0.append((r0, l0))

        cp_b.wait()
        cp_c = load_x(lax.rem(my + 1, N_DEV), 0)
        buf_b[:, :] = jnp.dot(
            xbuf[1], w_ref[:, :], preferred_element_type=jnp.float32)
        cp_c.wait()
        cp_d = load_x(lax.rem(my + 2, N_DEV), 1)
        buf_c[:, :] = jnp.dot(
            xbuf[0], w_ref[:, :], preferred_element_type=jnp.float32)

        rs1 = []
        for sub in range(2):
            r0, l0 = rs0[sub]
            r0.wait()
            l0.wait()
            rs_ = slice(sub * hh, (sub + 1) * hh)
            buf_b[rs_, 0:nh] = buf_b[rs_, 0:nh] + comm_r[0, rs_, :]
            buf_c[rs_, nh:n] = buf_c[rs_, nh:n] + comm_l[0, rs_, :]
            r1 = rdma_r(buf_b.at[sub_rows(sub), half_cols(0)],
                        comm_r.at[1, sub_rows(sub), :], 2 + sub)
            l1 = rdma_l(buf_c.at[sub_rows(sub), half_cols(1)],
                        comm_l.at[1, sub_rows(sub), :], 2 + sub)
            r1.start()
            l1.start()
            rs1.append((r1, l1))

        cp_d.wait()
        buf_a[:, :] = jnp.dot(
            xbuf[1], w_ref[:, :], preferred_element_type=jnp.float32)

        rs2 = []
        for sub in range(2):
            r1, l1 = rs1[sub]
            r1.wait()
            l1.wait()
            rs_ = slice(sub * hh, (sub + 1) * hh)
            buf_a[rs_, 0:nh] = buf_a[rs_, 0:nh] + comm_r[1, rs_, :]
            buf_a[rs_, nh:n] = buf_a[rs_, nh:n] + comm_l[1, rs_, :]
            r2 = rdma_r(buf_a.at[sub_rows(sub), half_cols(0)],
                        comm_r.at[0, sub_rows(sub), :], sub)
            l2 = rdma_l(buf_a.at[sub_rows(sub), half_cols(1)],
                        comm_l.at[0, sub_rows(sub), :], sub)
            r2.start()
            l2.start()
            rs2.append((r2, l2))

        q_r = lax.rem(my + 1, N_DEV)
        q_l = lax.rem(my - 1 + N_DEV, N_DEV)
        ag0 = []
        for sub in range(2):
            r2, l2 = rs2[sub]
            r2.wait()
            l2.wait()
            rs_ = slice(sub * hh, (sub + 1) * hh)
            buf_c[rs_, 0:nh] = buf_c[rs_, 0:nh] + comm_r[0, rs_, :]
            buf_b[rs_, nh:n] = buf_b[rs_, nh:n] + comm_l[0, rs_, :]
            ar = rdma_r(buf_c.at[sub_rows(sub), half_cols(0)],
                        comm_r.at[1, sub_rows(sub), :], 2 + sub)
            al = rdma_l(buf_b.at[sub_rows(sub), half_cols(1)],
                        comm_l.at[1, sub_rows(sub), :], 2 + sub)
            ar.start()
            al.start()
            ag0.append((ar, al))
        st_r = pltpu.make_async_copy(
            buf_c.at[:, half_cols(0)],
            out_ref.at[pl.ds(q_r * ch, ch), half_cols(0)], cps.at[0])
        st_l = pltpu.make_async_copy(
            buf_b.at[:, half_cols(1)],
            out_ref.at[pl.ds(q_l * ch, ch), half_cols(1)], cps.at[1])
        st_r.start()
        st_l.start()

        out_cps = []
        ag1 = []
        for sub in range(2):
            ar0, al0 = ag0[sub]
            ar0.wait()
            al0.wait()
            cpr = pltpu.make_async_copy(
                comm_r.at[1, sub_rows(sub), :], out_at(my, 0, sub),
                cpo.at[sub])
            cpl = pltpu.make_async_copy(
                comm_l.at[1, sub_rows(sub), :], out_at(my, 1, sub),
                cpo.at[2 + sub])
            cpr.start()
            cpl.start()
            out_cps += [cpr, cpl]
            ar = rdma_r(comm_r.at[1, sub_rows(sub), :],
                        comm_r.at[0, sub_rows(sub), :], sub)
            al = rdma_l(comm_l.at[1, sub_rows(sub), :],
                        comm_l.at[0, sub_rows(sub), :], sub)
            ar.start()
            al.start()
            ag1.append((ar, al))

        c_r2 = lax.rem(my - 1 + N_DEV, N_DEV)
        c_l2 = lax.rem(my + 1, N_DEV)
        ag2 = []
        for sub in range(2):
            ar1, al1 = ag1[sub]
            ar1.wait()
            al1.wait()
            cpr = pltpu.make_async_copy(
                comm_r.at[0, sub_rows(sub), :], out_at(c_r2, 0, sub),
                cpo.at[4 + sub])
            cpl = pltpu.make_async_copy(
                comm_l.at[0, sub_rows(sub), :], out_at(c_l2, 1, sub),
                cpo.at[6 + sub])
            cpr.start()
            cpl.start()
            out_cps += [cpr, cpl]
            ar = rdma_r(comm_r.at[0, sub_rows(sub), :],
                        out_at(c_r2, 0, sub), 2 + sub)
            al = rdma_l(comm_l.at[0, sub_rows(sub), :],
                        out_at(c_l2, 1, sub), 2 + sub)
            ar.start()
            al.start()
            ag2.append((ar, al))

        for ar, al in ag2:
            ar.wait()
            al.wait()
        for cp in out_cps:
            cp.wait()
        st_r.wait()
        st_l.wait()

    return pl.pallas_call(
        body,
        out_shape=jax.ShapeDtypeStruct((m, n), jnp.float32),
        in_specs=[
            pl.BlockSpec(memory_space=pl.ANY),
            pl.BlockSpec(memory_space=pltpu.MemorySpace.VMEM),
            pl.BlockSpec(memory_space=pl.ANY),
        ],
        out_specs=pl.BlockSpec(memory_space=pl.ANY),
        input_output_aliases={2: 0},
        scratch_shapes=[
            pltpu.VMEM((2, ch, k_per), jnp.float32),
            pltpu.VMEM((ch, n), jnp.float32),
            pltpu.VMEM((ch, n), jnp.float32),
            pltpu.VMEM((ch, n), jnp.float32),
            pltpu.VMEM((2, ch, nh), jnp.float32),
            pltpu.VMEM((2, ch, nh), jnp.float32),
            pltpu.SemaphoreType.DMA((4,)),
            pltpu.SemaphoreType.DMA((4,)),
            pltpu.SemaphoreType.DMA((4,)),
            pltpu.SemaphoreType.DMA((4,)),
            pltpu.SemaphoreType.DMA((2,)),
            pltpu.SemaphoreType.DMA((8,)),
        ],
        compiler_params=pltpu.CompilerParams(
            collective_id=0, vmem_limit_bytes=60 * 1024 * 1024
        ),
    )(x, w_mat, jnp.zeros((m, n), jnp.float32))


# device time: 304902 ns/iter; 1.0378x vs baseline; 1.0378x over previous
import jax
import jax.numpy as jnp
from jax import lax
from jax.experimental import pallas as pl
from jax.experimental.pallas import tpu as pltpu

N_DEV = 4


def kernel(x, w_mat):
    m, k_per = x.shape
    _, n = w_mat.shape
    ch = m // N_DEV
    nh = n // 2
    hh = ch // 2

    def body(x_ref, w_ref, out_ref, xbuf, buf_a, buf_b, buf_c,
             comm_r, comm_l, send_r, recv_r, send_l, recv_l, cps, cpo):
        my = lax.axis_index("i")
        left = lax.rem(my + N_DEV - 1, N_DEV)
        right = lax.rem(my + 1, N_DEV)

        def sub_rows(sub):
            return pl.ds(sub * hh, hh)

        def half_cols(half):
            return pl.ds(half * nh, nh)

        def out_at(c, half, sub):
            return out_ref.at[pl.ds(c * ch + sub * hh, hh), half_cols(half)]

        def rdma_r(src, dst, slot):
            return pltpu.make_async_remote_copy(
                src_ref=src, dst_ref=dst,
                send_sem=send_r.at[slot], recv_sem=recv_r.at[slot],
                device_id=(right,), device_id_type=pl.DeviceIdType.MESH,
            )

        def rdma_l(src, dst, slot):
            return pltpu.make_async_remote_copy(
                src_ref=src, dst_ref=dst,
                send_sem=send_l.at[slot], recv_sem=recv_l.at[slot],
                device_id=(left,), device_id_type=pl.DeviceIdType.MESH,
            )

        def load_x(c, slot):
            cp = pltpu.make_async_copy(
                x_ref.at[pl.ds(c * ch, ch), :], xbuf.at[slot], cps.at[slot])
            cp.start()
            return cp

        cp_a = load_x(my, 0)

        barrier_sem = pltpu.get_barrier_semaphore()
        for nbr in (left, right):
            pl.semaphore_signal(
                barrier_sem, inc=1,
                device_id=(nbr,), device_id_type=pl.DeviceIdType.MESH,
            )
        pl.semaphore_wait(barrier_sem, 2)

        cp_a.wait()
        cp_b = load_x(lax.rem(my - 1 + N_DEV, N_DEV), 1)
        rs0 = []
        for sub in range(2):
            buf_a[sub * hh:(sub + 1) * hh, :] = jnp.dot(
                xbuf[0, sub * hh:(sub + 1) * hh, :], w_ref[:, :],
                preferred_element_type=jnp.float32)
            r0 = rdma_r(buf_a.at[sub_rows(sub), half_cols(0)],
                        comm_r.at[0, sub_rows(sub), :], sub)
            l0 = rdma_l(buf_a.at[sub_rows(sub), half_cols(1)],
                        comm_l.at[0, sub_rows(sub), :], sub)
            r0.start()
            l0.start()
            rs0.append((r0, l0))

        cp_b.wait()
        cp_c = load_x(lax.rem(my + 1, N_DEV), 0)
        buf_b[:, :] = jnp.dot(
            xbuf[1], w_ref[:, :], preferred_element_type=jnp.float32)
        cp_c.wait()
        cp_d = load_x(lax.rem(my + 2, N_DEV), 1)
        buf_c[:, :] = jnp.dot(
            xbuf[0], w_ref[:, :], preferred_element_type=jnp.float32)

        rs1 = []
        for sub in range(2):
            r0, l0 = rs0[sub]
            r0.wait()
            l0.wait()
            rs_ = slice(sub * hh, (sub + 1) * hh)
            buf_b[rs_, 0:nh] = buf_b[rs_, 0:nh] + comm_r[0, rs_, :]
            buf_c[rs_, nh:n] = buf_c[rs_, nh:n] + comm_l[0, rs_, :]
            r1 = rdma_r(buf_b.at[sub_rows(sub), half_cols(0)],
                        comm_r.at[1, sub_rows(sub), :], 2 + sub)
            l1 = rdma_l(buf_c.at[sub_rows(sub), half_cols(1)],
                        comm_l.at[1, sub_rows(sub), :], 2 + sub)
            r1.start()
            l1.start()
            rs1.append((r1, l1))

        cp_d.wait()
        buf_a[:, :] = jnp.dot(
            xbuf[1], w_ref[:, :], preferred_element_type=jnp.float32)

        rs2 = []
        for sub in range(2):
            r1, l1 = rs1[sub]
            r1.wait()
            l1.wait()
            rs_ = slice(sub * hh, (sub + 1) * hh)
            buf_a[rs_, 0:nh] = buf_a[rs_, 0:nh] + comm_r[1, rs_, :]
            buf_a[rs_, nh:n] = buf_a[rs_, nh:n] + comm_l[1, rs_, :]
            r2 = rdma_r(buf_a.at[sub_rows(sub), half_cols(0)],
                        comm_r.at[0, sub_rows(sub), :], sub)
            l2 = rdma_l(buf_a.at[sub_rows(sub), half_cols(1)],
                        comm_l.at[0, sub_rows(sub), :], sub)
            r2.start()
            l2.start()
            rs2.append((r2, l2))

        q_r = lax.rem(my + 1, N_DEV)
        q_l = lax.rem(my - 1 + N_DEV, N_DEV)
        ag0 = []
        for sub in range(2):
            r2, l2 = rs2[sub]
            r2.wait()
            l2.wait()
            rs_ = slice(sub * hh, (sub + 1) * hh)
            buf_c[rs_, 0:nh] = buf_c[rs_, 0:nh] + comm_r[0, rs_, :]
            buf_b[rs_, nh:n] = buf_b[rs_, nh:n] + comm_l[0, rs_, :]
            ar = rdma_r(buf_c.at[sub_rows(sub), half_cols(0)],
                        comm_r.at[1, sub_rows(sub), :], 2 + sub)
            al = rdma_l(buf_b.at[sub_rows(sub), half_cols(1)],
                        comm_l.at[1, sub_rows(sub), :], 2 + sub)
            ar.start()
            al.start()
            ag0.append((ar, al))
        st_r = pltpu.make_async_copy(
            buf_c.at[:, half_cols(0)],
            out_ref.at[pl.ds(q_r * ch, ch), half_cols(0)], cps.at[0])
        st_l = pltpu.make_async_copy(
            buf_b.at[:, half_cols(1)],
            out_ref.at[pl.ds(q_l * ch, ch), half_cols(1)], cps.at[1])
        st_r.start()
        st_l.start()

        out_cps = []
        ag1 = []
        for sub in range(2):
            ar0, al0 = ag0[sub]
            ar0.wait()
            al0.wait()
            cpr = pltpu.make_async_copy(
                comm_r.at[1, sub_rows(sub), :], out_at(my, 0, sub),
                cpo.at[sub])
            cpl = pltpu.make_async_copy(
                comm_l.at[1, sub_rows(sub), :], out_at(my, 1, sub),
                cpo.at[2 + sub])
            cpr.start()
            cpl.start()
            out_cps += [cpr, cpl]
            ar = rdma_r(comm_r.at[1, sub_rows(sub), :],
                        comm_r.at[0, sub_rows(sub), :], sub)
            al = rdma_l(comm_l.at[1, sub_rows(sub), :],
                        comm_l.at[0, sub_rows(sub), :], sub)
            ar.start()
            al.start()
            ag1.append((ar, al))

        c_r2 = lax.rem(my - 1 + N_DEV, N_DEV)
        c_l2 = lax.rem(my + 1, N_DEV)
        ag2 = []
        for sub in range(2):
            ar1, al1 = ag1[sub]
            ar1.wait()
            al1.wait()
            cpr = pltpu.make_async_copy(
                comm_r.at[0, sub_rows(sub), :], out_at(c_r2, 0, sub),
                cpo.at[4 + sub])
            cpl = pltpu.make_async_copy(
                comm_l.at[0, sub_rows(sub), :], out_at(c_l2, 1, sub),
                cpo.at[6 + sub])
            cpr.start()
            cpl.start()
            out_cps += [cpr, cpl]
            ar = rdma_r(comm_r.at[0, sub_rows(sub), :],
                        out_at(c_r2, 0, sub), 2 + sub)
            al = rdma_l(comm_l.at[0, sub_rows(sub), :],
                        out_at(c_l2, 1, sub), 2 + sub)
            ar.start()
            al.start()
            ag2.append((ar, al))

        for ar, al in ag2:
            ar.wait()
            al.wait()
        for cp in out_cps:
            cp.wait()
        st_r.wait()
        st_l.wait()

    return pl.pallas_call(
        body,
        out_shape=jax.ShapeDtypeStruct((m, n), jnp.float32),
        in_specs=[
            pl.BlockSpec(memory_space=pl.ANY),
            pl.BlockSpec(memory_space=pltpu.MemorySpace.VMEM),
        ],
        out_specs=pl.BlockSpec(memory_space=pl.ANY),
        scratch_shapes=[
            pltpu.VMEM((2, ch, k_per), jnp.float32),
            pltpu.VMEM((ch, n), jnp.float32),
            pltpu.VMEM((ch, n), jnp.float32),
            pltpu.VMEM((ch, n), jnp.float32),
            pltpu.VMEM((2, ch, nh), jnp.float32),
            pltpu.VMEM((2, ch, nh), jnp.float32),
            pltpu.SemaphoreType.DMA((4,)),
            pltpu.SemaphoreType.DMA((4,)),
            pltpu.SemaphoreType.DMA((4,)),
            pltpu.SemaphoreType.DMA((4,)),
            pltpu.SemaphoreType.DMA((2,)),
            pltpu.SemaphoreType.DMA((8,)),
        ],
        compiler_params=pltpu.CompilerParams(
            collective_id=0, vmem_limit_bytes=60 * 1024 * 1024
        ),
    )(x, w_mat)
